# baseline (speedup 1.0000x reference)
import functools

import jax
import jax.numpy as jnp
from jax import lax
from jax.experimental import pallas as pl
from jax.experimental.pallas import tpu as pltpu
from jax.experimental.pallas import tpu_sc as plsc

N_NODES = 10000
D = 256
DH = 128
N_TILES = 16
NB = 80
KB = 128
E_PAD = N_TILES * NB * KB
SB = 8
NSB = NB // SB
N_ACC = 10240
ROWS_PER_TILE = N_ACC // N_TILES
ZCHUNKS = ROWS_PER_TILE // KB

_mesh = plsc.VectorSubcoreMesh(core_axis_name="c", subcore_axis_name="s")


@functools.partial(
    pl.kernel,
    out_type=jax.ShapeDtypeStruct((2 * N_ACC, DH), jnp.float32),
    mesh=_mesh,
    scratch_types=[
        pltpu.VMEM((2, SB, KB), jnp.int32),
        pltpu.VMEM((2, SB, KB), jnp.int32),
        pltpu.VMEM((2, SB, KB), jnp.float32),
        pltpu.VMEM((KB, DH), jnp.float32),
        pltpu.VMEM((KB, DH), jnp.float32),
        pltpu.VMEM_SHARED((N_ACC, DH), jnp.float32),
        [pltpu.SemaphoreType.DMA] * 2,
        [pltpu.SemaphoreType.DMA] * 2,
    ],
)
def _sc_segment(xs_hbm, isrc_hbm, idst_hbm, w_hbm, out_hbm,
                isrc_v, idst_v, w_v, rows0, rows1, acc,
                gsems, isems):
    c = lax.axis_index("c")
    s = lax.axis_index("s")
    rbufs = (rows0, rows1)

    off = jnp.broadcast_to(c.astype(jnp.int32), (16,))
    zero = jnp.zeros((16,), jnp.float32)

    @pl.loop(0, KB)
    def _zr(e):
        for r in range(DH // 16):
            rows0[e, pl.ds(16 * r, 16)] = zero

    for j in range(ZCHUNKS):
        pltpu.sync_copy(rows0, acc.at[pl.ds((s * ZCHUNKS + j) * KB, KB)])
    plsc.subcore_barrier()

    def start_sb(g, slot):
        src = pl.ds(g * SB, SB)
        pltpu.async_copy(isrc_hbm.at[s, src], isrc_v.at[slot], isems[slot])
        pltpu.async_copy(idst_hbm.at[s, src], idst_v.at[slot], isems[slot])
        pltpu.async_copy(w_hbm.at[s, src], w_v.at[slot], isems[slot])

    def wait_sb(g, slot):
        src = pl.ds(g * SB, SB)
        pltpu.make_async_copy(
            isrc_hbm.at[s, src], isrc_v.at[slot], isems[slot]).wait()
        pltpu.make_async_copy(
            idst_hbm.at[s, src], idst_v.at[slot], isems[slot]).wait()
        pltpu.make_async_copy(
            w_hbm.at[s, src], w_v.at[slot], isems[slot]).wait()
        @pl.loop(0, SB)
        def _adj(bb):
            for q in range(KB // 16):
                sl = pl.ds(16 * q, 16)
                isrc_v[slot, bb, sl] = isrc_v[slot, bb, sl] * 2 + off

    def start_gather(slot, row, j):
        pltpu.async_copy(xs_hbm.at[isrc_v.at[slot, row]], rbufs[j], gsems[j])

    def wait_gather(slot, row, j):
        pltpu.make_async_copy(
            xs_hbm.at[isrc_v.at[slot, row]], rbufs[j], gsems[j]).wait()

    def sync_scatter(slot, row, j):
        pltpu.sync_copy(rbufs[j], acc.at[idst_v.at[slot, row]], add=True)

    def scale(slot, k, j):
        rows = rbufs[j]

        @pl.loop(0, KB // 16)
        def _sc(q):
            wvec = w_v[slot, k, pl.ds(16 * q, 16)]
            for i in range(16):
                wj = jnp.broadcast_to(wvec[i], (16,))
                e = 16 * q + i
                for u in range(DH // 16):
                    sl = pl.ds(16 * u, 16)
                    rows[e, sl] = rows[e, sl] * wj

    start_sb(0, 0)
    wait_sb(0, 0)
    start_sb(1, 1)
    start_gather(0, 0, 0)

    @pl.loop(0, NSB // 2)
    def _gg(gg):
        for g2 in range(2):
            slot = g2
            nslot = (g2 + 1) % 2
            for k in range(SB):
                j = k % 2
                nj = (k + 1) % 2
                wait_gather(slot, k, j)
                if k == SB - 2:
                    start_gather(slot, k + 1, nj)
                    if g2 == 0:
                        wait_sb(2 * gg + 1, nslot)
                    else:
                        @pl.when(gg < NSB // 2 - 1)
                        def _():
                            wait_sb(2 * gg + 2, nslot)
                elif k < SB - 1:
                    start_gather(slot, k + 1, nj)
                elif g2 == 0:
                    start_gather(nslot, 0, nj)
                else:
                    @pl.when(gg < NSB // 2 - 1)
                    def _():
                        start_gather(nslot, 0, nj)
                scale(slot, k, j)
                sync_scatter(slot, k, j)
                if k == 1:
                    if g2 == 0:
                        @pl.when(gg >= 1)
                        def _():
                            start_sb(2 * gg + 1, nslot)
                    else:
                        @pl.when(gg < NSB // 2 - 1)
                        def _():
                            start_sb(2 * gg + 2, nslot)

    plsc.subcore_barrier()

    base = c * N_ACC + s * ROWS_PER_TILE
    pltpu.sync_copy(acc.at[pl.ds(s * ROWS_PER_TILE, ROWS_PER_TILE)],
                    out_hbm.at[pl.ds(base, ROWS_PER_TILE)])


def _tc_body(xd_ref, glo_ref, ghi_ref, wlt_ref, wit_ref, out_ref):
    xd = xd_ref[...]
    g = jnp.concatenate([glo_ref[...], ghi_ref[...]], axis=1)
    y = jnp.dot(xd + g, wlt_ref[...], preferred_element_type=jnp.float32)
    y += jnp.dot(xd * g, wit_ref[...], preferred_element_type=jnp.float32)
    out_ref[...] = jnp.where(y >= 0, y, 0.01 * y)


_TR = 512


def _tc_post(xd, sc_out, wlt, wit):
    return pl.pallas_call(
        _tc_body,
        grid=(N_ACC // _TR,),
        in_specs=[
            pl.BlockSpec((_TR, D), lambda i: (i, 0)),
            pl.BlockSpec((_TR, DH), lambda i: (i, 0)),
            pl.BlockSpec((_TR, DH), lambda i: (i + N_ACC // _TR, 0)),
            pl.BlockSpec((D, D), lambda i: (0, 0)),
            pl.BlockSpec((D, D), lambda i: (0, 0)),
        ],
        out_specs=pl.BlockSpec((_TR, D), lambda i: (i, 0)),
        out_shape=jax.ShapeDtypeStruct((N_NODES, D), jnp.float32),
    )(xd, sc_out, sc_out, wlt, wit)


@jax.jit
def kernel(x_src, x_dst, edge_index, edge_weight, W_loop, W_intr):
    E = edge_index.shape[1]
    i_src = edge_index[0].astype(jnp.int32)
    i_dst = edge_index[1].astype(jnp.int32)
    w = edge_weight[:, 0]

    pad = E_PAD - E
    i_src_p = jnp.pad(i_src, (0, pad)).reshape(N_TILES, NB, KB)
    i_dst_p = jnp.pad(i_dst, (0, pad)).reshape(N_TILES, NB, KB)
    w_p = jnp.pad(w, (0, pad)).reshape(N_TILES, NB, KB)

    xs = x_src.reshape(2 * N_NODES, DH)

    out = _sc_segment(xs, i_src_p, i_dst_p, w_p)
    return _tc_post(x_dst, out, W_loop.T, W_intr.T)


# speedup vs baseline: 1.0012x; 1.0012x over previous
import functools

import jax
import jax.numpy as jnp
from jax import lax
from jax.experimental import pallas as pl
from jax.experimental.pallas import tpu as pltpu
from jax.experimental.pallas import tpu_sc as plsc

N_NODES = 10000
D = 256
DH = 128
N_TILES = 16
NB = 80
KB = 128
E_PAD = N_TILES * NB * KB
SB = 8
NSB = NB // SB
N_ACC = 10240
ROWS_PER_TILE = N_ACC // N_TILES
ZCHUNKS = ROWS_PER_TILE // KB

_sc_segment_built = None


def _sc_segment(*args):
    global _sc_segment_built
    if _sc_segment_built is None:
        _sc_segment_built = _build_sc_segment()
    return _sc_segment_built(*args)


def _build_sc_segment():
    mesh = plsc.VectorSubcoreMesh(core_axis_name="c", subcore_axis_name="s")
    return functools.partial(
        pl.kernel,
        out_type=jax.ShapeDtypeStruct((2 * N_ACC, DH), jnp.float32),
        mesh=mesh,
        scratch_types=[
            pltpu.VMEM((2, SB, KB), jnp.int32),
            pltpu.VMEM((2, SB, KB), jnp.int32),
            pltpu.VMEM((2, SB, KB), jnp.float32),
            pltpu.VMEM((KB, DH), jnp.float32),
            pltpu.VMEM((KB, DH), jnp.float32),
            pltpu.VMEM_SHARED((N_ACC, DH), jnp.float32),
            [pltpu.SemaphoreType.DMA] * 2,
            [pltpu.SemaphoreType.DMA] * 2,
        ],
    )(_sc_segment_body)


def _sc_segment_body(xs_hbm, isrc_hbm, idst_hbm, w_hbm, out_hbm,
                     isrc_v, idst_v, w_v, rows0, rows1, acc,
                     gsems, isems):
    c = lax.axis_index("c")
    s = lax.axis_index("s")
    rbufs = (rows0, rows1)

    off = jnp.broadcast_to(c.astype(jnp.int32), (16,))
    zero = jnp.zeros((16,), jnp.float32)

    @pl.loop(0, KB)
    def _zr(e):
        for r in range(DH // 16):
            rows0[e, pl.ds(16 * r, 16)] = zero

    for j in range(ZCHUNKS):
        pltpu.sync_copy(rows0, acc.at[pl.ds((s * ZCHUNKS + j) * KB, KB)])
    plsc.subcore_barrier()

    def start_sb(g, slot):
        src = pl.ds(g * SB, SB)
        pltpu.async_copy(isrc_hbm.at[s, src], isrc_v.at[slot], isems[slot])
        pltpu.async_copy(idst_hbm.at[s, src], idst_v.at[slot], isems[slot])
        pltpu.async_copy(w_hbm.at[s, src], w_v.at[slot], isems[slot])

    def wait_sb(g, slot):
        src = pl.ds(g * SB, SB)
        pltpu.make_async_copy(
            isrc_hbm.at[s, src], isrc_v.at[slot], isems[slot]).wait()
        pltpu.make_async_copy(
            idst_hbm.at[s, src], idst_v.at[slot], isems[slot]).wait()
        pltpu.make_async_copy(
            w_hbm.at[s, src], w_v.at[slot], isems[slot]).wait()
        @pl.loop(0, SB)
        def _adj(bb):
            for q in range(KB // 16):
                sl = pl.ds(16 * q, 16)
                isrc_v[slot, bb, sl] = isrc_v[slot, bb, sl] * 2 + off

    def start_gather(slot, row, j):
        pltpu.async_copy(xs_hbm.at[isrc_v.at[slot, row]], rbufs[j], gsems[j])

    def wait_gather(slot, row, j):
        pltpu.make_async_copy(
            xs_hbm.at[isrc_v.at[slot, row]], rbufs[j], gsems[j]).wait()

    def sync_scatter(slot, row, j):
        pltpu.sync_copy(rbufs[j], acc.at[idst_v.at[slot, row]], add=True)

    def scale(slot, k, j):
        rows = rbufs[j]

        @pl.loop(0, KB // 16)
        def _sc(q):
            wvec = w_v[slot, k, pl.ds(16 * q, 16)]
            for i in range(16):
                wj = jnp.broadcast_to(wvec[i], (16,))
                e = 16 * q + i
                for u in range(DH // 16):
                    sl = pl.ds(16 * u, 16)
                    rows[e, sl] = rows[e, sl] * wj

    start_sb(0, 0)
    wait_sb(0, 0)
    start_sb(1, 1)
    start_gather(0, 0, 0)

    @pl.loop(0, NSB // 2)
    def _gg(gg):
        for g2 in range(2):
            slot = g2
            nslot = (g2 + 1) % 2
            for k in range(SB):
                j = k % 2
                nj = (k + 1) % 2
                wait_gather(slot, k, j)
                if k == SB - 2:
                    start_gather(slot, k + 1, nj)
                    if g2 == 0:
                        wait_sb(2 * gg + 1, nslot)
                    else:
                        @pl.when(gg < NSB // 2 - 1)
                        def _():
                            wait_sb(2 * gg + 2, nslot)
                elif k < SB - 1:
                    start_gather(slot, k + 1, nj)
                elif g2 == 0:
                    start_gather(nslot, 0, nj)
                else:
                    @pl.when(gg < NSB // 2 - 1)
                    def _():
                        start_gather(nslot, 0, nj)
                scale(slot, k, j)
                sync_scatter(slot, k, j)
                if k == 1:
                    if g2 == 0:
                        @pl.when(gg >= 1)
                        def _():
                            start_sb(2 * gg + 1, nslot)
                    else:
                        @pl.when(gg < NSB // 2 - 1)
                        def _():
                            start_sb(2 * gg + 2, nslot)

    plsc.subcore_barrier()

    base = c * N_ACC + s * ROWS_PER_TILE
    pltpu.sync_copy(acc.at[pl.ds(s * ROWS_PER_TILE, ROWS_PER_TILE)],
                    out_hbm.at[pl.ds(base, ROWS_PER_TILE)])


def _tc_body(xd_ref, glo_ref, ghi_ref, wlt_ref, wit_ref, out_ref):
    xd = xd_ref[...]
    g = jnp.concatenate([glo_ref[...], ghi_ref[...]], axis=1)
    y = jnp.dot(xd + g, wlt_ref[...], preferred_element_type=jnp.float32)
    y += jnp.dot(xd * g, wit_ref[...], preferred_element_type=jnp.float32)
    out_ref[...] = jnp.where(y >= 0, y, 0.01 * y)


_TR = 512


def _tc_post(xd, sc_out, wlt, wit):
    return pl.pallas_call(
        _tc_body,
        grid=(N_ACC // _TR,),
        in_specs=[
            pl.BlockSpec((_TR, D), lambda i: (i, 0)),
            pl.BlockSpec((_TR, DH), lambda i: (i, 0)),
            pl.BlockSpec((_TR, DH), lambda i: (i + N_ACC // _TR, 0)),
            pl.BlockSpec((D, D), lambda i: (0, 0)),
            pl.BlockSpec((D, D), lambda i: (0, 0)),
        ],
        out_specs=pl.BlockSpec((_TR, D), lambda i: (i, 0)),
        out_shape=jax.ShapeDtypeStruct((N_NODES, D), jnp.float32),
    )(xd, sc_out, sc_out, wlt, wit)


@jax.jit
def kernel(x_src, x_dst, edge_index, edge_weight, W_loop, W_intr):
    E = edge_index.shape[1]
    i_src = edge_index[0].astype(jnp.int32)
    i_dst = edge_index[1].astype(jnp.int32)
    w = edge_weight[:, 0]

    pad = E_PAD - E
    i_src_p = jnp.pad(i_src, (0, pad)).reshape(N_TILES, NB, KB)
    i_dst_p = jnp.pad(i_dst, (0, pad)).reshape(N_TILES, NB, KB)
    w_p = jnp.pad(w, (0, pad)).reshape(N_TILES, NB, KB)

    xs = x_src.reshape(2 * N_NODES, DH)

    out = _sc_segment(xs, i_src_p, i_dst_p, w_p)
    return _tc_post(x_dst, out, W_loop.T, W_intr.T)
